# Initial kernel scaffold; baseline (speedup 1.0000x reference)
import jax
import jax.numpy as jnp
from jax.experimental import pallas as pl


def kernel(indices, table):
    raise NotImplementedError("write your pallas kernel here")


# speedup vs baseline: 2.7522x; 2.7522x over previous
import jax
import jax.numpy as jnp
from jax import lax
from jax.experimental import pallas as pl
from jax.experimental.pallas import tpu as pltpu
from jax.experimental.pallas import tpu_sc as plsc

BATCH = 4096
HIST = 50
EMBED = 128
TOTAL = BATCH * HIST
NC = 2
NS = 16
NW = NC * NS
B_PER_W = TOTAL // NW
CH = 128
N_CHUNKS = B_PER_W // CH


def _gather_body(idx_hbm, table_hbm, out_hbm, idx_v, rows_v, sem):
    wid = lax.axis_index("s") * NC + lax.axis_index("c")
    wbase = wid * B_PER_W

    def body(j, carry):
        base = wbase + j * CH
        pltpu.sync_copy(idx_hbm.at[pl.ds(base, CH)], idx_v)
        pltpu.async_copy(table_hbm.at[idx_v], rows_v, sem).wait()
        pltpu.sync_copy(rows_v, out_hbm.at[pl.ds(base, CH)])
        return carry

    lax.fori_loop(0, N_CHUNKS, body, 0)


def kernel(indices, table):
    flat_idx = indices.reshape(TOTAL)
    mesh = plsc.VectorSubcoreMesh(core_axis_name="c", subcore_axis_name="s")
    k = pl.kernel(
        _gather_body,
        mesh=mesh,
        out_type=jax.ShapeDtypeStruct((TOTAL, EMBED), jnp.float32),
        scratch_types=[
            pltpu.VMEM((CH,), jnp.int32),
            pltpu.VMEM((CH, EMBED), jnp.float32),
            pltpu.SemaphoreType.DMA,
        ],
    )
    out = k(flat_idx, table)
    return out.reshape(BATCH, HIST, EMBED)


# speedup vs baseline: 3.3338x; 1.2113x over previous
import jax
import jax.numpy as jnp
from jax import lax
from jax.experimental import pallas as pl
from jax.experimental.pallas import tpu as pltpu
from jax.experimental.pallas import tpu_sc as plsc

BATCH = 4096
HIST = 50
EMBED = 128
TOTAL = BATCH * HIST
NC = 2
NS = 16
NW = NC * NS
B_PER_W = TOTAL // NW
CH = 128
N_CHUNKS = B_PER_W // CH
NBUF = 2
N_OUTER = N_CHUNKS // NBUF


def _gather_body(idx_hbm, table_hbm, out_hbm, idx_v,
                 rows0, rows1, gsem0, gsem1, ssem0, ssem1):
    wid = lax.axis_index("s") * NC + lax.axis_index("c")
    wbase = wid * B_PER_W
    rows = (rows0, rows1)
    gsem = (gsem0, gsem1)
    ssem = (ssem0, ssem1)

    pltpu.sync_copy(idx_hbm.at[pl.ds(wbase, B_PER_W)], idx_v)

    def gather_start(j, b):
        pltpu.async_copy(
            table_hbm.at[idx_v.at[pl.ds(j * CH, CH)]], rows[b], gsem[b])

    for b in range(NBUF):
        gather_start(b, b)

    def body(g, carry):
        j0 = g * NBUF
        for b in range(NBUF):
            j = j0 + b
            pltpu.make_async_copy(
                table_hbm.at[idx_v.at[pl.ds(0, CH)]], rows[b], gsem[b]).wait()
            pltpu.async_copy(
                rows[b], out_hbm.at[pl.ds(wbase + j * CH, CH)], ssem[b])
            pltpu.make_async_copy(
                rows[b], out_hbm.at[pl.ds(wbase, CH)], ssem[b]).wait()

            @pl.when(j + NBUF < N_CHUNKS)
            def _():
                gather_start(j + NBUF, b)
        return carry

    lax.fori_loop(0, N_OUTER, body, 0)


def kernel(indices, table):
    flat_idx = indices.reshape(TOTAL)
    mesh = plsc.VectorSubcoreMesh(core_axis_name="c", subcore_axis_name="s")
    k = pl.kernel(
        _gather_body,
        mesh=mesh,
        out_type=jax.ShapeDtypeStruct((TOTAL, EMBED), jnp.float32),
        scratch_types=[
            pltpu.VMEM((B_PER_W,), jnp.int32),
            pltpu.VMEM((CH, EMBED), jnp.float32),
            pltpu.VMEM((CH, EMBED), jnp.float32),
            pltpu.SemaphoreType.DMA,
            pltpu.SemaphoreType.DMA,
            pltpu.SemaphoreType.DMA,
            pltpu.SemaphoreType.DMA,
        ],
    )
    out = k(flat_idx, table)
    return out.reshape(BATCH, HIST, EMBED)


# speedup vs baseline: 3.3382x; 1.0013x over previous
import jax
import jax.numpy as jnp
from jax import lax
from jax.experimental import pallas as pl
from jax.experimental.pallas import tpu as pltpu
from jax.experimental.pallas import tpu_sc as plsc

BATCH = 4096
HIST = 50
EMBED = 128
TOTAL = BATCH * HIST
NC = 2
NS = 16
NW = NC * NS
B_PER_W = TOTAL // NW
CH = 320
N_CHUNKS = B_PER_W // CH
NBUF = 2
N_OUTER = N_CHUNKS // NBUF


def _gather_body(idx_hbm, table_hbm, out_hbm, idx_v,
                 rows0, rows1, gsem0, gsem1, ssem0, ssem1):
    wid = lax.axis_index("s") * NC + lax.axis_index("c")
    wbase = wid * B_PER_W
    rows = (rows0, rows1)
    gsem = (gsem0, gsem1)
    ssem = (ssem0, ssem1)

    pltpu.sync_copy(idx_hbm.at[pl.ds(wbase, B_PER_W)], idx_v)

    def gather_start(j, b):
        pltpu.async_copy(
            table_hbm.at[idx_v.at[pl.ds(j * CH, CH)]], rows[b], gsem[b])

    for b in range(NBUF):
        gather_start(b, b)

    def body(g, carry):
        j0 = g * NBUF
        for b in range(NBUF):
            j = j0 + b
            pltpu.make_async_copy(
                table_hbm.at[idx_v.at[pl.ds(0, CH)]], rows[b], gsem[b]).wait()
            pltpu.async_copy(
                rows[b], out_hbm.at[pl.ds(wbase + j * CH, CH)], ssem[b])
            pltpu.make_async_copy(
                rows[b], out_hbm.at[pl.ds(wbase, CH)], ssem[b]).wait()

            @pl.when(j + NBUF < N_CHUNKS)
            def _():
                gather_start(j + NBUF, b)
        return carry

    lax.fori_loop(0, N_OUTER, body, 0)


def kernel(indices, table):
    flat_idx = indices.reshape(TOTAL)
    mesh = plsc.VectorSubcoreMesh(core_axis_name="c", subcore_axis_name="s")
    k = pl.kernel(
        _gather_body,
        mesh=mesh,
        out_type=jax.ShapeDtypeStruct((TOTAL, EMBED), jnp.float32),
        scratch_types=[
            pltpu.VMEM((B_PER_W,), jnp.int32),
            pltpu.VMEM((CH, EMBED), jnp.float32),
            pltpu.VMEM((CH, EMBED), jnp.float32),
            pltpu.SemaphoreType.DMA,
            pltpu.SemaphoreType.DMA,
            pltpu.SemaphoreType.DMA,
            pltpu.SemaphoreType.DMA,
        ],
    )
    out = k(flat_idx, table)
    return out.reshape(BATCH, HIST, EMBED)


# speedup vs baseline: 3.3443x; 1.0018x over previous
import jax
import jax.numpy as jnp
from jax import lax
from jax.experimental import pallas as pl
from jax.experimental.pallas import tpu as pltpu
from jax.experimental.pallas import tpu_sc as plsc

BATCH = 4096
HIST = 50
EMBED = 128
TOTAL = BATCH * HIST
NC = 2
NS = 16
NW = NC * NS
B_PER_W = TOTAL // NW
CH = 128
N_CHUNKS = B_PER_W // CH
NBUF = 5
LOOK = 3
N_OUTER = N_CHUNKS // NBUF


def _gather_body(idx_hbm, table_hbm, out_hbm, idx_v,
                 rows0, rows1, rows2, rows3, rows4,
                 gsem0, gsem1, gsem2, gsem3, gsem4,
                 ssem0, ssem1, ssem2, ssem3, ssem4):
    wid = lax.axis_index("s") * NC + lax.axis_index("c")
    wbase = wid * B_PER_W
    rows = (rows0, rows1, rows2, rows3, rows4)
    gsem = (gsem0, gsem1, gsem2, gsem3, gsem4)
    ssem = (ssem0, ssem1, ssem2, ssem3, ssem4)

    pltpu.sync_copy(idx_hbm.at[pl.ds(wbase, B_PER_W)], idx_v)

    def gather_start(j, b):
        pltpu.async_copy(
            table_hbm.at[idx_v.at[pl.ds(j * CH, CH)]], rows[b], gsem[b])

    def process(j, b):
        pltpu.make_async_copy(
            table_hbm.at[idx_v.at[pl.ds(0, CH)]], rows[b], gsem[b]).wait()
        pltpu.async_copy(
            rows[b], out_hbm.at[pl.ds(wbase + j * CH, CH)], ssem[b])

    def drain_store(b):
        pltpu.make_async_copy(
            rows[b], out_hbm.at[pl.ds(wbase, CH)], ssem[b]).wait()

    for b in range(LOOK):
        gather_start(b, b)

    for b in range(NBUF):
        process(b, b)
        jn = b + LOOK
        bn = jn % NBUF
        if jn >= NBUF:
            drain_store(bn)
        gather_start(jn, bn)

    def body(g, carry):
        j0 = g * NBUF
        for b in range(NBUF):
            j = j0 + b
            process(j, b)
            bn = (b + LOOK) % NBUF

            @pl.when(j + LOOK < N_CHUNKS)
            def _():
                drain_store(bn)
                gather_start(j + LOOK, bn)
        return carry

    lax.fori_loop(1, N_OUTER, body, 0)

    for b in range(NBUF):
        drain_store(b)


def kernel(indices, table):
    flat_idx = indices.reshape(TOTAL)
    mesh = plsc.VectorSubcoreMesh(core_axis_name="c", subcore_axis_name="s")
    k = pl.kernel(
        _gather_body,
        mesh=mesh,
        out_type=jax.ShapeDtypeStruct((TOTAL, EMBED), jnp.float32),
        scratch_types=(
            [pltpu.VMEM((B_PER_W,), jnp.int32)]
            + [pltpu.VMEM((CH, EMBED), jnp.float32)] * NBUF
            + [pltpu.SemaphoreType.DMA] * (2 * NBUF)
        ),
    )
    out = k(flat_idx, table)
    return out.reshape(BATCH, HIST, EMBED)


# speedup vs baseline: 5.4876x; 1.6409x over previous
import jax
import jax.numpy as jnp
from jax import lax
from jax.experimental import pallas as pl
from jax.experimental.pallas import tpu as pltpu
from jax.experimental.pallas import tpu_sc as plsc

BATCH = 4096
HIST = 50
EMBED = 128
NC = 2
NS = 16
NW = NC * NS
BAT_PER_W = BATCH // NW
N_CHUNKS = BAT_PER_W
NBUF = 4
LOOK = 2
N_OUTER = N_CHUNKS // NBUF


def _gather_body(idx_hbm, table_hbm, out_hbm, idx_v,
                 rows0, rows1, rows2, rows3,
                 gsem0, gsem1, gsem2, gsem3,
                 ssem0, ssem1, ssem2, ssem3):
    wid = lax.axis_index("s") * NC + lax.axis_index("c")
    wbat = wid * BAT_PER_W
    rows = (rows0, rows1, rows2, rows3)
    gsem = (gsem0, gsem1, gsem2, gsem3)
    ssem = (ssem0, ssem1, ssem2, ssem3)

    pltpu.sync_copy(idx_hbm.at[pl.ds(wbat, BAT_PER_W)], idx_v)

    def gather_start(j, b):
        pltpu.async_copy(
            table_hbm.at[idx_v.at[j]], rows[b], gsem[b])

    def process(j, b):
        pltpu.make_async_copy(
            table_hbm.at[idx_v.at[0]], rows[b], gsem[b]).wait()
        pltpu.async_copy(
            rows[b], out_hbm.at[wbat + j], ssem[b])

    def drain_store(b):
        pltpu.make_async_copy(
            rows[b], out_hbm.at[wbat], ssem[b]).wait()

    for b in range(LOOK):
        gather_start(b, b)

    for b in range(NBUF):
        process(b, b)
        jn = b + LOOK
        bn = jn % NBUF
        if jn >= NBUF:
            drain_store(bn)
        gather_start(jn, bn)

    def body(g, carry):
        j0 = g * NBUF
        for b in range(NBUF):
            j = j0 + b
            process(j, b)
            bn = (b + LOOK) % NBUF

            @pl.when(j + LOOK < N_CHUNKS)
            def _():
                drain_store(bn)
                gather_start(j + LOOK, bn)
        return carry

    lax.fori_loop(1, N_OUTER, body, 0)

    for b in range(NBUF):
        drain_store(b)


def kernel(indices, table):
    mesh = plsc.VectorSubcoreMesh(core_axis_name="c", subcore_axis_name="s")
    k = pl.kernel(
        _gather_body,
        mesh=mesh,
        out_type=jax.ShapeDtypeStruct((BATCH, HIST, EMBED), jnp.float32),
        scratch_types=(
            [pltpu.VMEM((BAT_PER_W, HIST), jnp.int32)]
            + [pltpu.VMEM((HIST, EMBED), jnp.float32)] * NBUF
            + [pltpu.SemaphoreType.DMA] * (2 * NBUF)
        ),
    )
    return k(indices, table)


# speedup vs baseline: 5.8709x; 1.0699x over previous
import jax
import jax.numpy as jnp
from jax import lax
from jax.experimental import pallas as pl
from jax.experimental.pallas import tpu as pltpu
from jax.experimental.pallas import tpu_sc as plsc

BATCH = 4096
HIST = 50
EMBED = 128
TOTAL = BATCH * HIST
NC = 2
NS = 16
NW = NC * NS
BAT_PER_W = BATCH // NW
B_PER_W = BAT_PER_W * HIST
CHB = 4
CH = CHB * HIST
N_CHUNKS = BAT_PER_W // CHB
NBUF = 4
LOOK = 2
N_OUTER = N_CHUNKS // NBUF


def _gather_body(idx_hbm, table_hbm, out_hbm, idx_v,
                 rows0, rows1, rows2, rows3,
                 gsem0, gsem1, gsem2, gsem3,
                 ssem0, ssem1, ssem2, ssem3):
    wid = lax.axis_index("s") * NC + lax.axis_index("c")
    wbase = wid * B_PER_W
    wbat = wid * BAT_PER_W
    rows = (rows0, rows1, rows2, rows3)
    gsem = (gsem0, gsem1, gsem2, gsem3)
    ssem = (ssem0, ssem1, ssem2, ssem3)

    pltpu.sync_copy(idx_hbm.at[pl.ds(wbase, B_PER_W)], idx_v)

    def gather_start(j, b):
        pltpu.async_copy(
            table_hbm.at[idx_v.at[pl.ds(j * CH, CH)]], rows[b], gsem[b])

    def process(j, b):
        pltpu.make_async_copy(
            table_hbm.at[idx_v.at[pl.ds(0, CH)]], rows[b], gsem[b]).wait()
        for k in range(CHB):
            pltpu.async_copy(
                rows[b].at[pl.ds(k * HIST, HIST)],
                out_hbm.at[wbat + j * CHB + k], ssem[b])

    def drain_store(b):
        for _ in range(CHB):
            pltpu.make_async_copy(
                rows[b].at[pl.ds(0, HIST)], out_hbm.at[wbat], ssem[b]).wait()

    for b in range(LOOK):
        gather_start(b, b)

    for b in range(NBUF):
        process(b, b)
        jn = b + LOOK
        bn = jn % NBUF
        if jn >= NBUF:
            drain_store(bn)
        gather_start(jn, bn)

    def body(g, carry):
        j0 = g * NBUF
        for b in range(NBUF):
            j = j0 + b
            process(j, b)
            bn = (b + LOOK) % NBUF

            @pl.when(j + LOOK < N_CHUNKS)
            def _():
                drain_store(bn)
                gather_start(j + LOOK, bn)
        return carry

    lax.fori_loop(1, N_OUTER, body, 0)

    for b in range(NBUF):
        drain_store(b)


def kernel(indices, table):
    flat_idx = indices.reshape(TOTAL)
    mesh = plsc.VectorSubcoreMesh(core_axis_name="c", subcore_axis_name="s")
    k = pl.kernel(
        _gather_body,
        mesh=mesh,
        out_type=jax.ShapeDtypeStruct((BATCH, HIST, EMBED), jnp.float32),
        scratch_types=(
            [pltpu.VMEM((B_PER_W,), jnp.int32)]
            + [pltpu.VMEM((CH, EMBED), jnp.float32)] * NBUF
            + [pltpu.SemaphoreType.DMA] * (2 * NBUF)
        ),
    )
    return k(flat_idx, table)
